# baseline (speedup 1.0000x reference)
import jax
import jax.numpy as jnp
from jax import lax
from jax.experimental import pallas as pl
from jax.experimental.pallas import tpu as pltpu
from jax.experimental.pallas import tpu_sc as plsc

X, Y, Z = 192, 160, 192
N = 300
NUM_CORES = 2
NUM_TILES = 16
NUM_WORKERS = NUM_CORES * NUM_TILES
SLOTS = 10
FULL_SLOTS = 9
REM = N - FULL_SLOTS * NUM_WORKERS
NPADRR = SLOTS * NUM_WORKERS
L = 16
G = 3 * X * (Y // 8)
CHUNKS = 3 * L


def _tre_body(f3_hbm, flt_hbm, mlt_hbm, sp_hbm, out_hbm,
              fl_v, ml_v, sp_v, idx_v, buf_v,
              part_v, all_v, out_v, shared, sem):
    c = lax.axis_index("c")
    s = lax.axis_index("s")
    wid = c * NUM_TILES + s

    pltpu.sync_copy(flt_hbm, fl_v)
    pltpu.sync_copy(mlt_hbm, ml_v)
    pltpu.sync_copy(sp_hbm, sp_v)

    slots = lax.iota(jnp.int32, L)
    idxn = jnp.minimum(slots * NUM_WORKERS + wid, NPADRR - 1)
    fxj = plsc.load_gather(fl_v, [idxn])
    fyj = plsc.load_gather(fl_v, [idxn + NPADRR])
    fzj = plsc.load_gather(fl_v, [idxn + 2 * NPADRR])
    mxj = plsc.load_gather(ml_v, [idxn])
    myj = plsc.load_gather(ml_v, [idxn + NPADRR])
    mzj = plsc.load_gather(ml_v, [idxn + 2 * NPADRR])

    idx_v[pl.ds(0, L)] = fxj * (Y // 8) + jnp.right_shift(fyj, 3)
    idx_v[pl.ds(L, L)] = jnp.bitwise_and(fyj, 7)
    idx_v[pl.ds(2 * L, L)] = jnp.right_shift(fzj, 7) * 128

    cnt = jnp.where(wid < REM, SLOTS, FULL_SLOTS)
    offs = jnp.minimum(slots * L, 2 * L)

    def issue(i, carry):
        v = plsc.load_gather(idx_v, [offs + i])
        g = v[0]
        iy = v[1]
        zoff = pl.multiple_of(v[2], 128)
        for ch in range(3):
            pltpu.async_copy(
                f3_hbm.at[g + ch * (X * Y // 8), iy, pl.ds(zoff, 128)],
                buf_v.at[ch * L + i], sem)
        return carry

    lax.fori_loop(0, cnt, issue, 0)

    def drain(i, carry):
        pltpu.make_async_copy(
            f3_hbm.at[0, 0, pl.ds(0, 128)], buf_v.at[0], sem).wait()
        return carry

    lax.fori_loop(0, 3 * cnt, drain, 0)

    lanes = jnp.bitwise_and(fzj, 127)
    dispx = plsc.load_gather(buf_v, [slots, lanes])
    dispy = plsc.load_gather(buf_v, [slots + L, lanes])
    dispz = plsc.load_gather(buf_v, [slots + 2 * L, lanes])
    sx = sp_v[pl.ds(0, L)]
    sy = sp_v[pl.ds(L, L)]
    sz = sp_v[pl.ds(2 * L, L)]
    dx = (fxj.astype(jnp.float32) + dispx - mxj) * sx
    dy = (fyj.astype(jnp.float32) + dispy - myj) * sy
    dz = (fzj.astype(jnp.float32) + dispz - mzj) * sz
    d2 = dx * dx + dy * dy + dz * dz
    n_global = slots * NUM_WORKERS + wid
    acc = jnp.where(n_global < N, d2, 0.0)

    part_v[...] = acc
    pltpu.sync_copy(part_v, shared.at[pl.ds(s * L, L)])
    plsc.subcore_barrier()

    @pl.when(s == 0)
    def _():
        pltpu.sync_copy(shared, all_v)

        def red(r, tot):
            return tot + plsc.load_gather(all_v, [r * L + slots])

        tot = lax.fori_loop(0, NUM_TILES, red, jnp.zeros((L,), jnp.float32))
        total = tot[0]
        for i in range(1, L):
            total = total + tot[i]
        out_v[...] = jnp.full((L,), total * (1.0 / N), jnp.float32)
        pltpu.sync_copy(out_v, out_hbm.at[pl.ds(c * 32, L)])


@jax.jit
def _tre(f3, flt, mlt, spb):
    mesh = plsc.VectorSubcoreMesh(core_axis_name="c", subcore_axis_name="s")
    run = pl.kernel(
        _tre_body,
        out_type=jax.ShapeDtypeStruct((64,), jnp.float32),
        mesh=mesh,
        scratch_types=[
            pltpu.VMEM((3 * NPADRR,), jnp.int32),
            pltpu.VMEM((3 * NPADRR,), jnp.float32),
            pltpu.VMEM((3 * L,), jnp.float32),
            pltpu.VMEM((3 * L,), jnp.int32),
            pltpu.VMEM((CHUNKS, 128), jnp.float32),
            pltpu.VMEM((L,), jnp.float32),
            pltpu.VMEM((NUM_TILES * L,), jnp.float32),
            pltpu.VMEM((L,), jnp.float32),
            pltpu.VMEM_SHARED((NUM_TILES * L,), jnp.float32),
            pltpu.SemaphoreType.DMA,
        ],
        compiler_params=pltpu.CompilerParams(
            use_tc_tiling_on_sc=True, needs_layout_passes=False),
    )
    return run(f3, flt, mlt, spb)


def kernel(vector_field, moving_landmarks, fixed_landmarks, image_spacing):
    f3 = vector_field.reshape(G, 8, Z)
    flt = jnp.pad(fixed_landmarks[0].astype(jnp.int32).T,
                  ((0, 0), (0, NPADRR - N))).reshape(3 * NPADRR)
    mlt = jnp.pad(moving_landmarks[0].astype(jnp.float32).T,
                  ((0, 0), (0, NPADRR - N))).reshape(3 * NPADRR)
    spb = jnp.repeat(image_spacing.astype(jnp.float32), L)
    out = _tre(f3, flt, mlt, spb)
    return out[0] + out[32]


# speedup vs baseline: 1.3408x; 1.3408x over previous
import jax
import jax.numpy as jnp
from jax import lax
from jax.experimental import pallas as pl
from jax.experimental.pallas import tpu as pltpu
from jax.experimental.pallas import tpu_sc as plsc

X, Y, Z = 192, 160, 192
N = 300
NUM_TILES = 16
SLOTS = 19
FULL_SLOTS = 18
REM = N - FULL_SLOTS * NUM_TILES
NPADRR = SLOTS * NUM_TILES
L = 16
G = 3 * X * (Y // 8)
SLOT_STRIDE = 32
CHUNKS = 3 * SLOT_STRIDE
OFF_ML = 3 * NPADRR
OFF_SP = 6 * NPADRR
AUX_W = OFF_SP + 3 * L


def _tre_body(f3_hbm, aux_hbm, out_hbm,
              aux_v, idx_v, buf_v, part_v, all_v, out_v, shared, sem):
    s = lax.axis_index("s")

    pltpu.sync_copy(aux_hbm, aux_v)

    lanes16 = lax.iota(jnp.int32, L)
    fx = [None, None]
    fy = [None, None]
    fz = [None, None]
    mx = [None, None]
    my = [None, None]
    mz = [None, None]
    for q in range(2):
        slots = q * L + lanes16
        idxn = jnp.minimum(slots * NUM_TILES + s, NPADRR - 1)
        fx[q] = plsc.load_gather(aux_v, [idxn])
        fy[q] = plsc.load_gather(aux_v, [idxn + NPADRR])
        fz[q] = plsc.load_gather(aux_v, [idxn + 2 * NPADRR])
        mx[q] = plsc.bitcast(
            plsc.load_gather(aux_v, [idxn + OFF_ML]), jnp.float32)
        my[q] = plsc.bitcast(
            plsc.load_gather(aux_v, [idxn + OFF_ML + NPADRR]), jnp.float32)
        mz[q] = plsc.bitcast(
            plsc.load_gather(aux_v, [idxn + OFF_ML + 2 * NPADRR]),
            jnp.float32)
        idx_v[pl.ds(q * L, L)] = (
            fx[q] * (Y // 8) + jnp.right_shift(fy[q], 3))
        idx_v[pl.ds(SLOT_STRIDE + q * L, L)] = jnp.bitwise_and(fy[q], 7)
        idx_v[pl.ds(2 * SLOT_STRIDE + q * L, L)] = (
            jnp.right_shift(fz[q], 7) * 128)

    cnt = jnp.where(s < REM, SLOTS, FULL_SLOTS)
    offs = jnp.minimum(lanes16 * SLOT_STRIDE, 2 * SLOT_STRIDE)

    def issue(i, carry):
        v = plsc.load_gather(idx_v, [offs + i])
        g = v[0]
        iy = v[1]
        zoff = pl.multiple_of(v[2], 128)
        for ch in range(3):
            pltpu.async_copy(
                f3_hbm.at[g + ch * (X * Y // 8), iy, pl.ds(zoff, 128)],
                buf_v.at[ch * SLOT_STRIDE + i], sem)
        return carry

    lax.fori_loop(0, cnt, issue, 0)

    def drain(i, carry):
        pltpu.make_async_copy(
            f3_hbm.at[0, 0, pl.ds(0, 128)], buf_v.at[0], sem).wait()
        return carry

    lax.fori_loop(0, 3 * cnt, drain, 0)

    sx = plsc.bitcast(aux_v[pl.ds(OFF_SP, L)], jnp.float32)
    sy = plsc.bitcast(aux_v[pl.ds(OFF_SP + L, L)], jnp.float32)
    sz = plsc.bitcast(aux_v[pl.ds(OFF_SP + 2 * L, L)], jnp.float32)
    acc = jnp.zeros((L,), jnp.float32)
    for q in range(2):
        slots = q * L + lanes16
        rows = jnp.minimum(slots, SLOT_STRIDE - 1)
        zlane = jnp.bitwise_and(fz[q], 127)
        dispx = plsc.load_gather(buf_v, [rows, zlane])
        dispy = plsc.load_gather(buf_v, [rows + SLOT_STRIDE, zlane])
        dispz = plsc.load_gather(buf_v, [rows + 2 * SLOT_STRIDE, zlane])
        dx = (fx[q].astype(jnp.float32) + dispx - mx[q]) * sx
        dy = (fy[q].astype(jnp.float32) + dispy - my[q]) * sy
        dz = (fz[q].astype(jnp.float32) + dispz - mz[q]) * sz
        d2 = dx * dx + dy * dy + dz * dz
        n_global = slots * NUM_TILES + s
        acc = acc + jnp.where(n_global < N, d2, 0.0)

    part_v[...] = acc
    pltpu.sync_copy(part_v, shared.at[pl.ds(s * L, L)])
    plsc.subcore_barrier()

    @pl.when(s == 0)
    def _():
        pltpu.sync_copy(shared, all_v)

        def red(r, tot):
            return tot + plsc.load_gather(all_v, [r * L + lanes16])

        tot = lax.fori_loop(0, NUM_TILES, red, jnp.zeros((L,), jnp.float32))
        total = tot[0]
        for i in range(1, L):
            total = total + tot[i]
        out_v[...] = jnp.full((L,), total * (1.0 / N), jnp.float32)
        pltpu.sync_copy(out_v, out_hbm)


@jax.jit
def _tre(f3, aux):
    mesh = plsc.VectorSubcoreMesh(
        core_axis_name="c", subcore_axis_name="s", num_cores=1)
    run = pl.kernel(
        _tre_body,
        out_type=jax.ShapeDtypeStruct((L,), jnp.float32),
        mesh=mesh,
        scratch_types=[
            pltpu.VMEM((AUX_W,), jnp.int32),
            pltpu.VMEM((CHUNKS,), jnp.int32),
            pltpu.VMEM((CHUNKS, 128), jnp.float32),
            pltpu.VMEM((L,), jnp.float32),
            pltpu.VMEM((NUM_TILES * L,), jnp.float32),
            pltpu.VMEM((L,), jnp.float32),
            pltpu.VMEM_SHARED((NUM_TILES * L,), jnp.float32),
            pltpu.SemaphoreType.DMA,
        ],
        compiler_params=pltpu.CompilerParams(
            use_tc_tiling_on_sc=True, needs_layout_passes=False),
    )
    return run(f3, aux)


def kernel(vector_field, moving_landmarks, fixed_landmarks, image_spacing):
    f3 = vector_field.reshape(G, 8, Z)
    flt = jnp.pad(fixed_landmarks[0].astype(jnp.int32).T,
                  ((0, 0), (0, NPADRR - N))).reshape(3 * NPADRR)
    mlt = jax.lax.bitcast_convert_type(
        jnp.pad(moving_landmarks[0].astype(jnp.float32).T,
                ((0, 0), (0, NPADRR - N))), jnp.int32).reshape(3 * NPADRR)
    spb = jnp.repeat(
        jax.lax.bitcast_convert_type(image_spacing.astype(jnp.float32),
                                     jnp.int32), L)
    aux = jnp.concatenate([flt, mlt, spb])
    out = _tre(f3, aux)
    return out[0]
